# baseline (speedup 1.0000x reference)
import functools

import jax
import jax.numpy as jnp
from jax import lax
from jax.experimental import pallas as pl
from jax.experimental.pallas import tpu as pltpu
from jax.experimental.pallas import tpu_sc as plsc

B = 4096
T = 200
D = 128
NBUF = 3

_INFO = plsc.get_sparse_core_info()
NC = _INFO.num_cores
NS = _INFO.num_subcores
NW = NC * NS
BPW = B // NW

_MESH = plsc.VectorSubcoreMesh(core_axis_name="c", subcore_axis_name="s")


@functools.partial(
    pl.kernel,
    mesh=_MESH,
    out_type=jax.ShapeDtypeStruct((B, T, D), jnp.float32),
    scratch_types=[
        pltpu.VMEM((BPW,), jnp.int32),
        pltpu.VMEM((BPW, D), jnp.float32),
        pltpu.VMEM((NBUF, T, D), jnp.float32),
    ] + [pltpu.SemaphoreType.DMA] * (NBUF + 1),
)
def _sc_expand(t_hbm, table_hbm, out_hbm, idx_v, rows_v, exp_v, gsem, *sems):
    wid = lax.axis_index("s") * NC + lax.axis_index("c")
    base = wid * BPW
    pltpu.sync_copy(t_hbm.at[pl.ds(base, BPW)], idx_v)
    pltpu.async_copy(table_hbm.at[idx_v], rows_v, gsem).wait()

    def _build(buf, b):
        vecs = [rows_v[b, pl.ds(k * 16, 16)] for k in range(D // 16)]

        def fill(j2, _):
            for u in range(2):
                for k in range(D // 16):
                    exp_v[buf, j2 * 2 + u, pl.ds(k * 16, 16)] = vecs[k]
            return 0

        lax.fori_loop(0, T // 2, fill, 0)

    def _start(buf, b):
        pltpu.async_copy(exp_v.at[buf], out_hbm.at[base + b], sems[buf])

    def _wait(buf):
        pltpu.make_async_copy(exp_v.at[buf], out_hbm.at[base], sems[buf]).wait()

    def body(grp, _):
        for buf in range(NBUF):
            b = grp * NBUF + buf

            @pl.when(grp > 0)
            def _():
                _wait(buf)

            _build(buf, b)
            _start(buf, b)
        return 0

    lax.fori_loop(0, BPW // NBUF, body, 0)
    for i in range(BPW % NBUF):
        b = (BPW // NBUF) * NBUF + i
        _wait(i)
        _build(i, b)
        _start(i, b)
    for buf in range(NBUF):
        _wait(buf)


def kernel(t, n_tokens, table):
    del n_tokens
    return _sc_expand(t, table)


# speedup vs baseline: 1.2372x; 1.2372x over previous
import jax
import jax.numpy as jnp
from jax import lax
from jax.experimental import pallas as pl

B = 4096
T = 200
D = 128
V = 60

BB = 64
GRID = B // BB


def _tc_body(t_ref, table_ref, out_ref):
    idx = t_ref[0, 0, :]
    onehot = (idx[:, None] == lax.broadcasted_iota(jnp.int32, (BB, V), 1)
              ).astype(jnp.float32)
    emb = jnp.dot(onehot, table_ref[...],
                  preferred_element_type=jnp.float32,
                  precision=lax.Precision.HIGHEST)
    out_ref[...] = jnp.broadcast_to(emb[:, None, :], (BB, T, D))


@jax.jit
def _run(t, table):
    t3 = t.reshape(GRID, 1, BB)
    return pl.pallas_call(
        _tc_body,
        grid=(GRID,),
        in_specs=[
            pl.BlockSpec((1, 1, BB), lambda i: (i, 0, 0)),
            pl.BlockSpec((V, D), lambda i: (0, 0)),
        ],
        out_specs=pl.BlockSpec((BB, T, D), lambda i: (i, 0, 0)),
        out_shape=jax.ShapeDtypeStruct((B, T, D), jnp.float32),
    )(t3, table)


def kernel(t, n_tokens, table):
    del n_tokens
    return _run(t, table)
